# baseline (speedup 1.0000x reference)
import functools

import jax
import jax.numpy as jnp
from jax.experimental import pallas as pl
from jax.experimental.pallas import tpu as pltpu

_BF16 = jnp.bfloat16
_NN = (((1,), (0,)), ((), ()))
_NT = (((1,), (1,)), ((), ()))
_TN = (((0,), (0,)), ((), ()))


def _mish(v):
    u = jnp.exp(jnp.minimum(v, 20.0))
    w = u * (u + 2.0)
    return v * w * jax.lax.reciprocal(w + 2.0)


def _fused_kernel(xt_ref, ft_ref, att_ref, w1t_ref, b1_ref,
                  w2_ref, b2_ref, ww_ref, bw_ref, wa_ref, ba_ref,
                  wo1_ref, bo1_ref, wo2_ref, bo2_ref,
                  ow_ref, oa_ref,
                  ftb, atb, w1x, exb, w2b, wwa, woz):
    f32 = jnp.float32

    @pl.when(pl.program_id(0) == 0)
    def _prep():
        ftb[...] = ft_ref[:].T.astype(_BF16)
        atb[...] = att_ref[:].astype(_BF16)
        w1x[...] = jnp.zeros_like(w1x)
        for s in range(8):
            w1x[112 * s:112 * s + 100, :] = (
                w1t_ref[100 * s:100 * s + 100, :].astype(_BF16))
        w1x[896:912, :] = w1t_ref[800:816, :].astype(_BF16)
        exb[...] = jnp.zeros_like(exb)
        w2b[...] = w2_ref[:].astype(_BF16)
        wwa[0:128, :] = ww_ref[:].astype(_BF16)
        wwa[128:256, :] = wa_ref[:].astype(_BF16)
        woz[...] = jnp.zeros_like(woz)
        woz[0:1, 0:128] = wo1_ref[:]
        woz[1:2, 128:256] = wo2_ref[:]

    xt = xt_ref[:]
    BB = xt.shape[1]
    exb[896:912, :] = xt[8:24, :].astype(_BF16)

    iota_f = jax.lax.broadcasted_iota(jnp.int32, (38, 1), 0).astype(f32)
    iota_a = jax.lax.broadcasted_iota(jnp.int32, (241, 1), 0).astype(f32)
    idx_f = xt[0:2, :].reshape(1, 2 * BB)
    idx_a = xt[2:8, :].reshape(1, 6 * BB)
    oht_f = (idx_f == iota_f).astype(_BF16)
    oht_a = (idx_a == iota_a).astype(_BF16)
    embt_f = jax.lax.dot_general(ftb[...], oht_f, _NN,
                                 preferred_element_type=f32)
    embt_a = jax.lax.dot_general(atb[...], oht_a, _NN,
                                 preferred_element_type=f32)
    exb[0:100, :] = embt_f[:, 0:BB].astype(_BF16)
    exb[112:212, :] = embt_f[:, BB:2 * BB].astype(_BF16)
    for s in range(6):
        exb[112 * (s + 2):112 * (s + 2) + 100, :] = (
            embt_a[:, s * BB:(s + 1) * BB].astype(_BF16))

    h1 = jax.lax.dot_general(exb[...], w1x[...], _TN,
                             preferred_element_type=f32)
    h1 = jnp.tanh(h1 + b1_ref[:]).astype(_BF16)
    h2 = _mish(jax.lax.dot_general(h1, w2b[...], _NT,
                                   preferred_element_type=f32) + b2_ref[:])
    h2 = h2.astype(_BF16)
    bwa = jnp.concatenate([bw_ref[:], ba_ref[:]])
    hwa = _mish(jax.lax.dot_general(h2, wwa[...], _NT,
                                    preferred_element_type=f32) + bwa)
    o2 = jax.lax.dot_general(woz[...], hwa, _NT,
                             preferred_element_type=f32)
    ow_ref[:] = jax.nn.sigmoid(o2[0:1, :] + bo1_ref[:])
    oa_ref[:] = jnp.tanh(o2[1:2, :] + bo2_ref[:])


@functools.partial(jax.jit, static_argnames=("block_b",))
def _run(xt, fighter_table, ability_table_t, W1t, b1, W2, b2,
         Wwin, bwin, Wadv, badv, Wo1, bo1, Wo2, bo2, block_b=4096):
    B = xt.shape[1]
    rep = lambda *shape: pl.BlockSpec(shape, lambda i: (0,) * len(shape))
    in_specs = [
        pl.BlockSpec((24, block_b), lambda i: (0, i)),
        rep(38, 100), rep(100, 241), rep(816, 512), rep(512),
        rep(256, 512), rep(256), rep(128, 256), rep(128),
        rep(128, 256), rep(128),
        rep(1, 128), rep(1), rep(1, 128), rep(1),
    ]
    scratch_shapes = [
        pltpu.VMEM((100, 38), _BF16), pltpu.VMEM((100, 241), _BF16),
        pltpu.VMEM((912, 512), _BF16), pltpu.VMEM((912, block_b), _BF16),
        pltpu.VMEM((256, 512), _BF16),
        pltpu.VMEM((256, 256), _BF16), pltpu.VMEM((2, 256), jnp.float32),
    ]
    out = pl.pallas_call(
        _fused_kernel,
        grid=(B // block_b,),
        in_specs=in_specs,
        out_specs=[pl.BlockSpec((1, block_b), lambda i: (0, i))] * 2,
        out_shape=[jax.ShapeDtypeStruct((1, B), jnp.float32)] * 2,
        scratch_shapes=scratch_shapes,
    )(xt, fighter_table, ability_table_t, W1t, b1, W2, b2,
      Wwin, bwin, Wadv, badv, Wo1, bo1, Wo2, bo2)
    return tuple(out)


def kernel(x, fighter_table, ability_table, W1, b1, W2, b2,
           Wwin, bwin, Wadv, badv, Wo1, bo1, Wo2, bo2):
    ow, oa = _run(x.T, fighter_table, ability_table.T, W1.T, b1, W2, b2,
                  Wwin, bwin, Wadv, badv, Wo1, bo1, Wo2, bo2)
    return (ow.reshape(-1, 1), oa.reshape(-1, 1))


# speedup vs baseline: 1.0101x; 1.0101x over previous
import functools

import jax
import jax.numpy as jnp
from jax.experimental import pallas as pl
from jax.experimental.pallas import tpu as pltpu

_BF16 = jnp.bfloat16
_NN = (((1,), (0,)), ((), ()))
_NT = (((1,), (1,)), ((), ()))
_TN = (((0,), (0,)), ((), ()))


def _mish(v):
    u = jnp.exp(jnp.minimum(v, 20.0))
    w = u * (u + 2.0)
    return v * w * jax.lax.reciprocal(w + 2.0)


def _fused_kernel(xt_ref, ft_ref, att_ref, w1t_ref, b1_ref,
                  w2_ref, b2_ref, ww_ref, bw_ref, wa_ref, ba_ref,
                  wo1_ref, bo1_ref, wo2_ref, bo2_ref,
                  ow_ref, oa_ref,
                  ftb, atb, w1x, exb, w2b, wwb, wab):
    f32 = jnp.float32

    @pl.when(pl.program_id(0) == 0)
    def _prep():
        ftb[...] = ft_ref[:].T.astype(_BF16)
        atb[...] = att_ref[:].astype(_BF16)
        w1x[...] = jnp.zeros_like(w1x)
        for s in range(8):
            w1x[112 * s:112 * s + 100, :] = (
                w1t_ref[100 * s:100 * s + 100, :].astype(_BF16))
        w1x[896:912, :] = w1t_ref[800:816, :].astype(_BF16)
        exb[...] = jnp.zeros_like(exb)
        w2b[...] = w2_ref[:].astype(_BF16)
        wwb[...] = ww_ref[:].astype(_BF16)
        wab[...] = wa_ref[:].astype(_BF16)

    xt = xt_ref[:]
    BB = xt.shape[1]
    exb[896:912, :] = xt[8:24, :].astype(_BF16)

    iota_f = jax.lax.broadcasted_iota(jnp.int32, (38, 1), 0).astype(f32)
    iota_a = jax.lax.broadcasted_iota(jnp.int32, (241, 1), 0).astype(f32)
    idx_f = xt[0:2, :].reshape(1, 2 * BB)
    idx_a = xt[2:8, :].reshape(1, 6 * BB)
    oht_f = (idx_f == iota_f).astype(_BF16)
    oht_a = (idx_a == iota_a).astype(_BF16)
    embt_f = jax.lax.dot_general(ftb[...], oht_f, _NN,
                                 preferred_element_type=f32)
    embt_a = jax.lax.dot_general(atb[...], oht_a, _NN,
                                 preferred_element_type=f32)
    exb[0:100, :] = embt_f[:, 0:BB].astype(_BF16)
    exb[112:212, :] = embt_f[:, BB:2 * BB].astype(_BF16)
    for s in range(6):
        exb[112 * (s + 2):112 * (s + 2) + 100, :] = (
            embt_a[:, s * BB:(s + 1) * BB].astype(_BF16))

    h1 = jax.lax.dot_general(exb[...], w1x[...], _TN,
                             preferred_element_type=f32)
    h1 = jnp.tanh(h1 + b1_ref[:]).astype(_BF16)
    h2 = _mish(jax.lax.dot_general(h1, w2b[...], _NT,
                                   preferred_element_type=f32) + b2_ref[:])
    h2 = h2.astype(_BF16)
    hw = _mish(jax.lax.dot_general(h2, wwb[...], _NT,
                                   preferred_element_type=f32) + bw_ref[:])
    ha = _mish(jax.lax.dot_general(h2, wab[...], _NT,
                                   preferred_element_type=f32) + ba_ref[:])
    ow = jax.lax.dot_general(wo1_ref[:], hw, _NT,
                             preferred_element_type=f32) + bo1_ref[:]
    oa = jax.lax.dot_general(wo2_ref[:], ha, _NT,
                             preferred_element_type=f32) + bo2_ref[:]
    ow_ref[:] = jax.nn.sigmoid(ow)
    oa_ref[:] = jnp.tanh(oa)


@functools.partial(jax.jit, static_argnames=("block_b",))
def _run(xt, fighter_table, ability_table_t, W1t, b1, W2, b2,
         Wwin, bwin, Wadv, badv, Wo1, bo1, Wo2, bo2, block_b=4096):
    B = xt.shape[1]
    rep = lambda *shape: pl.BlockSpec(shape, lambda i: (0,) * len(shape))
    in_specs = [
        pl.BlockSpec((24, block_b), lambda i: (0, i)),
        rep(38, 100), rep(100, 241), rep(816, 512), rep(512),
        rep(256, 512), rep(256), rep(128, 256), rep(128),
        rep(128, 256), rep(128),
        rep(1, 128), rep(1), rep(1, 128), rep(1),
    ]
    scratch_shapes = [
        pltpu.VMEM((100, 38), _BF16), pltpu.VMEM((100, 241), _BF16),
        pltpu.VMEM((912, 512), _BF16), pltpu.VMEM((912, block_b), _BF16),
        pltpu.VMEM((256, 512), _BF16),
        pltpu.VMEM((128, 256), _BF16), pltpu.VMEM((128, 256), _BF16),
    ]
    out = pl.pallas_call(
        _fused_kernel,
        grid=(B // block_b,),
        in_specs=in_specs,
        out_specs=[pl.BlockSpec((1, block_b), lambda i: (0, i))] * 2,
        out_shape=[jax.ShapeDtypeStruct((1, B), jnp.float32)] * 2,
        scratch_shapes=scratch_shapes,
    )(xt, fighter_table, ability_table_t, W1t, b1, W2, b2,
      Wwin, bwin, Wadv, badv, Wo1, bo1, Wo2, bo2)
    return tuple(out)


def kernel(x, fighter_table, ability_table, W1, b1, W2, b2,
           Wwin, bwin, Wadv, badv, Wo1, bo1, Wo2, bo2):
    ow, oa = _run(x.T, fighter_table, ability_table.T, W1.T, b1, W2, b2,
                  Wwin, bwin, Wadv, badv, Wo1, bo1, Wo2, bo2)
    return (ow.reshape(-1, 1), oa.reshape(-1, 1))
